# baseline (speedup 1.0000x reference)
import functools

import jax
import jax.numpy as jnp
from jax import lax
from jax.experimental import pallas as pl
from jax.experimental.pallas import tpu as pltpu
import jax.experimental.pallas.tpu_sc as plsc

NPRED = 4096
NGT = 16384
CHUNK = 512
NSTEPS = NGT // CHUNK

NW = 32
BPW = NPRED // NW


def _argmin_body(predp_ref, predpt_ref, gtpt_full_ref, gtpt_ref, colf_ref,
                 idx_ref, t_ref, min_ref, s_ref):
    j = pl.program_id(0)

    @pl.when(j == 0)
    def _init():
        gsum = jnp.sum(gtpt_full_ref[...], axis=1, keepdims=True)
        psum = jnp.sum(predpt_ref[...], axis=1, keepdims=True)
        t_ref[...] = gsum / NGT - psum / NPRED
        min_ref[...] = jnp.full((NPRED, 1), jnp.inf, jnp.float32)
        idx_ref[...] = jnp.zeros((NPRED, 1), jnp.int32)

    t = t_ref[...]
    g = gtpt_ref[...]
    g0, g1, g2 = g[0:1, :], g[1:2, :], g[2:3, :]
    pq = predp_ref[...].astype(jnp.bfloat16).astype(jnp.float32)
    ps0 = pq[:, 0:1] + t[0:1, 0:1]
    ps1 = pq[:, 1:2] + t[1:2, 0:1]
    ps2 = pq[:, 2:3] + t[2:3, 0:1]
    a2 = ps0 * ps0 + ps1 * ps1 + ps2 * ps2
    b2 = g0 * g0 + g1 * g1 + g2 * g2
    ps = jnp.concatenate([ps0, ps1, ps2], axis=1)
    gb2 = (g * -2.0).astype(jnp.bfloat16)
    dn = (((1,), (0,)), ((), ()))
    ab2 = lax.dot_general(ps, gb2, dn, preferred_element_type=jnp.float32)
    u = b2 + ab2
    s_ref[...] = u
    min_u = jnp.min(u, axis=1, keepdims=True)
    m = jnp.maximum(min_u + a2, 0.0)
    thr = jnp.maximum(m - a2, min_u)
    candf = jnp.min(jnp.where(s_ref[...] <= thr, colf_ref[...],
                              jnp.float32(CHUNK)),
                    axis=1, keepdims=True)
    cand = candf.astype(jnp.int32) + j * CHUNK
    upd = m < min_ref[...]
    idx_ref[...] = jnp.where(upd, cand, idx_ref[...])
    min_ref[...] = jnp.where(upd, m, min_ref[...])


def _argmin_call(predp, predpt, gtpt):
    return pl.pallas_call(
        _argmin_body,
        grid=(NSTEPS,),
        in_specs=[
            pl.BlockSpec((NPRED, 3), lambda j: (0, 0)),
            pl.BlockSpec((3, NPRED), lambda j: (0, 0)),
            pl.BlockSpec((3, NGT), lambda j: (0, 0)),
            pl.BlockSpec((3, CHUNK), lambda j: (0, j)),
            pl.BlockSpec((1, CHUNK), lambda j: (0, 0)),
        ],
        out_specs=[
            pl.BlockSpec((NPRED, 1), lambda j: (0, 0)),
            pl.BlockSpec((3, 1), lambda j: (0, 0)),
            pl.BlockSpec((NPRED, 1), lambda j: (0, 0)),
        ],
        out_shape=[
            jax.ShapeDtypeStruct((NPRED, 1), jnp.int32),
            jax.ShapeDtypeStruct((3, 1), jnp.float32),
            jax.ShapeDtypeStruct((NPRED, 1), jnp.float32),
        ],
        scratch_shapes=[pltpu.VMEM((NPRED, CHUNK), jnp.float32)],
    )(predp, predpt, gtpt, gtpt,
      jnp.arange(CHUNK, dtype=jnp.float32).reshape(1, CHUNK))


@functools.cache
def _sc_gather_fn():
    mesh = plsc.VectorSubcoreMesh(
        core_axis_name="c", subcore_axis_name="s",
        num_cores=2, num_subcores=16)

    @functools.partial(
        pl.kernel,
        out_type=jax.ShapeDtypeStruct((NPRED, 128), jnp.float32),
        mesh=mesh,
        scratch_types=[
            pltpu.VMEM((BPW,), jnp.int32),
            pltpu.VMEM((BPW, 128), jnp.float32),
            pltpu.SemaphoreType.DMA,
        ],
    )
    def _sc_gather(table_hbm, idx_hbm, out_hbm, idx_v, rows_v, sem):
        wid = lax.axis_index("s") * 2 + lax.axis_index("c")
        base = wid * BPW
        pltpu.sync_copy(idx_hbm.at[pl.ds(base, BPW)], idx_v)
        pltpu.async_copy(table_hbm.at[idx_v], rows_v, sem).wait()
        pltpu.sync_copy(rows_v, out_hbm.at[pl.ds(base, BPW)])

    return _sc_gather


def _loss_body(pf_ref, mt_ref, t_ref, out_ref):
    pf = pf_ref[...]
    mt = mt_ref[...]
    t = t_ref[...]

    hsum = jnp.float32(0.0)
    for k in range(3):
        d = pf[:, k:k + 1] + t[k:k + 1, 0:1] - mt[:, k:k + 1]
        ad = jnp.abs(d)
        h = jnp.where(ad < 1.0, 0.5 * ad * ad, ad - 0.5)
        hsum = hsum + jnp.sum(h)
    huber = hsum / jnp.float32(NPRED * 3)
    reg = huber / 2000.0

    pn0, pn1, pn2 = pf[:, 3:4], pf[:, 4:5], pf[:, 5:6]
    gn0, gn1, gn2 = mt[:, 3:4], mt[:, 4:5], mt[:, 5:6]
    dp = jnp.maximum(jnp.sqrt(pn0 * pn0 + pn1 * pn1 + pn2 * pn2), 1e-5)
    dg = jnp.maximum(jnp.sqrt(gn0 * gn0 + gn1 * gn1 + gn2 * gn2), 1e-5)
    cos = (pn0 * gn0 + pn1 * gn1 + pn2 * gn2) / (dp * dg)
    norm_loss = 1.0 - jnp.sum(cos) / jnp.float32(NPRED)

    out_ref[...] = jnp.broadcast_to(reg + norm_loss, (1, 1))


def _loss_call(pred_feat, matched, t):
    return pl.pallas_call(
        _loss_body,
        out_shape=jax.ShapeDtypeStruct((1, 1), jnp.float32),
    )(pred_feat, matched, t)


def kernel(pred_feat, pred_decoder, input_data, gt_data):
    predp = pred_feat[:, :3]
    predpt = predp.T
    gtpt = gt_data[:, :3].T

    idx2d, t, _ = _argmin_call(predp, predpt, gtpt)
    idx = idx2d.reshape(NPRED)

    table = jnp.pad(gt_data, ((0, 0), (0, 122)))
    matched = _sc_gather_fn()(table, idx)

    out = _loss_call(pred_feat, matched, t)
    return out.reshape(1)


# speedup vs baseline: 1.0000x; 1.0000x over previous
import functools

import jax
import jax.numpy as jnp
from jax import lax
from jax.experimental import pallas as pl
from jax.experimental.pallas import tpu as pltpu
import jax.experimental.pallas.tpu_sc as plsc

NPRED = 4096
NGT = 16384
CHUNK = 1024
NSTEPS = NGT // CHUNK

NW = 32
BPW = NPRED // NW


def _argmin_body(predp_ref, predpt_ref, gtpt_full_ref, gtpt_ref, colf_ref,
                 idx_ref, t_ref, min_ref, s_ref):
    j = pl.program_id(0)

    @pl.when(j == 0)
    def _init():
        gsum = jnp.sum(gtpt_full_ref[...], axis=1, keepdims=True)
        psum = jnp.sum(predpt_ref[...], axis=1, keepdims=True)
        t_ref[...] = gsum / NGT - psum / NPRED
        min_ref[...] = jnp.full((NPRED, 1), jnp.inf, jnp.float32)
        idx_ref[...] = jnp.zeros((NPRED, 1), jnp.int32)

    t = t_ref[...]
    g = gtpt_ref[...]
    g0, g1, g2 = g[0:1, :], g[1:2, :], g[2:3, :]
    pq = predp_ref[...].astype(jnp.bfloat16).astype(jnp.float32)
    ps0 = pq[:, 0:1] + t[0:1, 0:1]
    ps1 = pq[:, 1:2] + t[1:2, 0:1]
    ps2 = pq[:, 2:3] + t[2:3, 0:1]
    a2 = ps0 * ps0 + ps1 * ps1 + ps2 * ps2
    b2 = g0 * g0 + g1 * g1 + g2 * g2
    ps = jnp.concatenate([ps0, ps1, ps2], axis=1)
    gb2 = (g * -2.0).astype(jnp.bfloat16)
    dn = (((1,), (0,)), ((), ()))
    ab2 = lax.dot_general(ps, gb2, dn, preferred_element_type=jnp.float32)
    u = b2 + ab2
    s_ref[...] = u
    min_u = jnp.min(u, axis=1, keepdims=True)
    m = jnp.maximum(min_u + a2, 0.0)
    thr = jnp.maximum(m - a2, min_u)
    candf = jnp.min(jnp.where(s_ref[...] <= thr, colf_ref[...],
                              jnp.float32(CHUNK)),
                    axis=1, keepdims=True)
    cand = candf.astype(jnp.int32) + j * CHUNK
    upd = m < min_ref[...]
    idx_ref[...] = jnp.where(upd, cand, idx_ref[...])
    min_ref[...] = jnp.where(upd, m, min_ref[...])


def _argmin_call(predp, predpt, gtpt):
    return pl.pallas_call(
        _argmin_body,
        grid=(NSTEPS,),
        in_specs=[
            pl.BlockSpec((NPRED, 3), lambda j: (0, 0)),
            pl.BlockSpec((3, NPRED), lambda j: (0, 0)),
            pl.BlockSpec((3, NGT), lambda j: (0, 0)),
            pl.BlockSpec((3, CHUNK), lambda j: (0, j)),
            pl.BlockSpec((1, CHUNK), lambda j: (0, 0)),
        ],
        out_specs=[
            pl.BlockSpec((NPRED, 1), lambda j: (0, 0)),
            pl.BlockSpec((3, 1), lambda j: (0, 0)),
            pl.BlockSpec((NPRED, 1), lambda j: (0, 0)),
        ],
        out_shape=[
            jax.ShapeDtypeStruct((NPRED, 1), jnp.int32),
            jax.ShapeDtypeStruct((3, 1), jnp.float32),
            jax.ShapeDtypeStruct((NPRED, 1), jnp.float32),
        ],
        scratch_shapes=[pltpu.VMEM((NPRED, CHUNK), jnp.float32)],
    )(predp, predpt, gtpt, gtpt,
      jnp.arange(CHUNK, dtype=jnp.float32).reshape(1, CHUNK))


@functools.cache
def _sc_gather_fn():
    mesh = plsc.VectorSubcoreMesh(
        core_axis_name="c", subcore_axis_name="s",
        num_cores=2, num_subcores=16)

    @functools.partial(
        pl.kernel,
        out_type=jax.ShapeDtypeStruct((NPRED, 128), jnp.float32),
        mesh=mesh,
        scratch_types=[
            pltpu.VMEM((BPW,), jnp.int32),
            pltpu.VMEM((BPW, 128), jnp.float32),
            pltpu.SemaphoreType.DMA,
        ],
    )
    def _sc_gather(table_hbm, idx_hbm, out_hbm, idx_v, rows_v, sem):
        wid = lax.axis_index("s") * 2 + lax.axis_index("c")
        base = wid * BPW
        pltpu.sync_copy(idx_hbm.at[pl.ds(base, BPW)], idx_v)
        pltpu.async_copy(table_hbm.at[idx_v], rows_v, sem).wait()
        pltpu.sync_copy(rows_v, out_hbm.at[pl.ds(base, BPW)])

    return _sc_gather


def _loss_body(pf_ref, mt_ref, t_ref, out_ref):
    pf = pf_ref[...]
    mt = mt_ref[...]
    t = t_ref[...]

    hsum = jnp.float32(0.0)
    for k in range(3):
        d = pf[:, k:k + 1] + t[k:k + 1, 0:1] - mt[:, k:k + 1]
        ad = jnp.abs(d)
        h = jnp.where(ad < 1.0, 0.5 * ad * ad, ad - 0.5)
        hsum = hsum + jnp.sum(h)
    huber = hsum / jnp.float32(NPRED * 3)
    reg = huber / 2000.0

    pn0, pn1, pn2 = pf[:, 3:4], pf[:, 4:5], pf[:, 5:6]
    gn0, gn1, gn2 = mt[:, 3:4], mt[:, 4:5], mt[:, 5:6]
    dp = jnp.maximum(jnp.sqrt(pn0 * pn0 + pn1 * pn1 + pn2 * pn2), 1e-5)
    dg = jnp.maximum(jnp.sqrt(gn0 * gn0 + gn1 * gn1 + gn2 * gn2), 1e-5)
    cos = (pn0 * gn0 + pn1 * gn1 + pn2 * gn2) / (dp * dg)
    norm_loss = 1.0 - jnp.sum(cos) / jnp.float32(NPRED)

    out_ref[...] = jnp.broadcast_to(reg + norm_loss, (1, 1))


def _loss_call(pred_feat, matched, t):
    return pl.pallas_call(
        _loss_body,
        out_shape=jax.ShapeDtypeStruct((1, 1), jnp.float32),
    )(pred_feat, matched, t)


def kernel(pred_feat, pred_decoder, input_data, gt_data):
    predp = pred_feat[:, :3]
    predpt = predp.T
    gtpt = gt_data[:, :3].T

    idx2d, t, _ = _argmin_call(predp, predpt, gtpt)
    idx = idx2d.reshape(NPRED)

    table = jnp.pad(gt_data, ((0, 0), (0, 122)))
    matched = _sc_gather_fn()(table, idx)

    out = _loss_call(pred_feat, matched, t)
    return out.reshape(1)


# speedup vs baseline: 1.0269x; 1.0269x over previous
import functools

import jax
import jax.numpy as jnp
from jax import lax
from jax.experimental import pallas as pl
from jax.experimental.pallas import tpu as pltpu
import jax.experimental.pallas.tpu_sc as plsc

NPRED = 4096
NGT = 16384
CHUNK = 512
NSTEPS = NGT // CHUNK

NW = 32
BPW = NPRED // NW


def _argmin_body(predp_ref, predpt_ref, gtpt_full_ref, gtpt_ref, colf_ref,
                 idx_ref, t_ref, min_ref, s_ref):
    j = pl.program_id(0)

    @pl.when(j == 0)
    def _init():
        gsum = jnp.sum(gtpt_full_ref[...], axis=1, keepdims=True)
        psum = jnp.sum(predpt_ref[...], axis=1, keepdims=True)
        t_ref[...] = gsum / NGT - psum / NPRED
        min_ref[...] = jnp.full((NPRED, 1), jnp.inf, jnp.float32)
        idx_ref[...] = jnp.zeros((NPRED, 1), jnp.int32)

    t = t_ref[...]
    g = gtpt_ref[...]
    g0, g1, g2 = g[0:1, :], g[1:2, :], g[2:3, :]
    pq = predp_ref[...].astype(jnp.bfloat16).astype(jnp.float32)
    ps0 = pq[:, 0:1] + t[0:1, 0:1]
    ps1 = pq[:, 1:2] + t[1:2, 0:1]
    ps2 = pq[:, 2:3] + t[2:3, 0:1]
    a2 = ps0 * ps0 + ps1 * ps1 + ps2 * ps2
    b2 = g0 * g0 + g1 * g1 + g2 * g2
    ps = jnp.concatenate([ps0, ps1, ps2], axis=1)
    gb2 = (g * -2.0).astype(jnp.bfloat16)
    dn = (((1,), (0,)), ((), ()))
    ab2 = lax.dot_general(ps, gb2, dn, preferred_element_type=jnp.float32)
    sp = (a2 + b2) + ab2
    s_ref[...] = sp
    m = jnp.maximum(jnp.min(sp, axis=1, keepdims=True), 0.0)
    candf = jnp.min(jnp.where(s_ref[...] <= m, colf_ref[...],
                              jnp.float32(CHUNK)),
                    axis=1, keepdims=True)
    cand = candf.astype(jnp.int32) + j * CHUNK
    upd = m < min_ref[...]
    idx_ref[...] = jnp.where(upd, cand, idx_ref[...])
    min_ref[...] = jnp.where(upd, m, min_ref[...])


def _argmin_call(predp, predpt, gtpt):
    return pl.pallas_call(
        _argmin_body,
        grid=(NSTEPS,),
        in_specs=[
            pl.BlockSpec((NPRED, 3), lambda j: (0, 0)),
            pl.BlockSpec((3, NPRED), lambda j: (0, 0)),
            pl.BlockSpec((3, NGT), lambda j: (0, 0)),
            pl.BlockSpec((3, CHUNK), lambda j: (0, j)),
            pl.BlockSpec((1, CHUNK), lambda j: (0, 0)),
        ],
        out_specs=[
            pl.BlockSpec((NPRED, 1), lambda j: (0, 0)),
            pl.BlockSpec((3, 1), lambda j: (0, 0)),
            pl.BlockSpec((NPRED, 1), lambda j: (0, 0)),
        ],
        out_shape=[
            jax.ShapeDtypeStruct((NPRED, 1), jnp.int32),
            jax.ShapeDtypeStruct((3, 1), jnp.float32),
            jax.ShapeDtypeStruct((NPRED, 1), jnp.float32),
        ],
        scratch_shapes=[pltpu.VMEM((NPRED, CHUNK), jnp.float32)],
    )(predp, predpt, gtpt, gtpt,
      jnp.arange(CHUNK, dtype=jnp.float32).reshape(1, CHUNK))


@functools.cache
def _sc_gather_fn():
    mesh = plsc.VectorSubcoreMesh(
        core_axis_name="c", subcore_axis_name="s",
        num_cores=2, num_subcores=16)

    @functools.partial(
        pl.kernel,
        out_type=jax.ShapeDtypeStruct((NPRED, 128), jnp.float32),
        mesh=mesh,
        scratch_types=[
            pltpu.VMEM((BPW,), jnp.int32),
            pltpu.VMEM((BPW, 128), jnp.float32),
            pltpu.SemaphoreType.DMA,
        ],
    )
    def _sc_gather(table_hbm, idx_hbm, out_hbm, idx_v, rows_v, sem):
        wid = lax.axis_index("s") * 2 + lax.axis_index("c")
        base = wid * BPW
        pltpu.sync_copy(idx_hbm.at[pl.ds(base, BPW)], idx_v)
        pltpu.async_copy(table_hbm.at[idx_v], rows_v, sem).wait()
        pltpu.sync_copy(rows_v, out_hbm.at[pl.ds(base, BPW)])

    return _sc_gather


def _loss_body(pf_ref, mt_ref, t_ref, out_ref):
    pf = pf_ref[...]
    mt = mt_ref[...]
    t = t_ref[...]

    hsum = jnp.float32(0.0)
    for k in range(3):
        d = pf[:, k:k + 1] + t[k:k + 1, 0:1] - mt[:, k:k + 1]
        ad = jnp.abs(d)
        h = jnp.where(ad < 1.0, 0.5 * ad * ad, ad - 0.5)
        hsum = hsum + jnp.sum(h)
    huber = hsum / jnp.float32(NPRED * 3)
    reg = huber / 2000.0

    pn0, pn1, pn2 = pf[:, 3:4], pf[:, 4:5], pf[:, 5:6]
    gn0, gn1, gn2 = mt[:, 3:4], mt[:, 4:5], mt[:, 5:6]
    dp = jnp.maximum(jnp.sqrt(pn0 * pn0 + pn1 * pn1 + pn2 * pn2), 1e-5)
    dg = jnp.maximum(jnp.sqrt(gn0 * gn0 + gn1 * gn1 + gn2 * gn2), 1e-5)
    cos = (pn0 * gn0 + pn1 * gn1 + pn2 * gn2) / (dp * dg)
    norm_loss = 1.0 - jnp.sum(cos) / jnp.float32(NPRED)

    out_ref[...] = jnp.broadcast_to(reg + norm_loss, (1, 1))


def _loss_call(pred_feat, matched, t):
    return pl.pallas_call(
        _loss_body,
        out_shape=jax.ShapeDtypeStruct((1, 1), jnp.float32),
    )(pred_feat, matched, t)


def kernel(pred_feat, pred_decoder, input_data, gt_data):
    predp = pred_feat[:, :3]
    predpt = predp.T
    gtpt = gt_data[:, :3].T

    idx2d, t, _ = _argmin_call(predp, predpt, gtpt)
    idx = idx2d.reshape(NPRED)

    table = jnp.pad(gt_data, ((0, 0), (0, 122)))
    matched = _sc_gather_fn()(table, idx)

    out = _loss_call(pred_feat, matched, t)
    return out.reshape(1)


# speedup vs baseline: 1.0423x; 1.0149x over previous
import functools

import jax
import jax.numpy as jnp
from jax import lax
from jax.experimental import pallas as pl
from jax.experimental.pallas import tpu as pltpu
import jax.experimental.pallas.tpu_sc as plsc

NPRED = 4096
NGT = 16384
CHUNK = 512
NSTEPS = NGT // CHUNK

NW = 32
BPW = NPRED // NW


def _argmin_body(predp_ref, predpt_ref, gtpt_full_ref, gtpt_ref, colf_ref,
                 idx_ref, t_ref, min_ref, s_ref):
    j = pl.program_id(0)

    @pl.when(j == 0)
    def _init():
        gsum = jnp.sum(gtpt_full_ref[...], axis=1, keepdims=True)
        psum = jnp.sum(predpt_ref[...], axis=1, keepdims=True)
        t_ref[...] = gsum / NGT - psum / NPRED
        min_ref[...] = jnp.full((NPRED, 1), jnp.inf, jnp.float32)
        idx_ref[...] = jnp.zeros((NPRED, 1), jnp.int32)

    t = t_ref[...]
    g = gtpt_ref[...]
    g0, g1, g2 = g[0:1, :], g[1:2, :], g[2:3, :]
    pq = predp_ref[...].astype(jnp.bfloat16).astype(jnp.float32)
    ps0 = pq[:, 0:1] + t[0:1, 0:1]
    ps1 = pq[:, 1:2] + t[1:2, 0:1]
    ps2 = pq[:, 2:3] + t[2:3, 0:1]
    a2 = ps0 * ps0 + ps1 * ps1 + ps2 * ps2
    b2 = g0 * g0 + g1 * g1 + g2 * g2
    ps = jnp.concatenate([ps0, ps1, ps2], axis=1)
    gb2 = (g * -2.0).astype(jnp.bfloat16)
    dn = (((1,), (0,)), ((), ()))
    ab2 = lax.dot_general(ps, gb2, dn, preferred_element_type=jnp.float32)
    sp = (a2 + b2) + ab2
    s_ref[...] = sp
    m = jnp.maximum(jnp.min(sp, axis=1, keepdims=True), 0.0)
    candf = jnp.min(jnp.where(s_ref[...] <= m, colf_ref[...],
                              jnp.float32(CHUNK)),
                    axis=1, keepdims=True)
    cand = candf.astype(jnp.int32) + j * CHUNK
    upd = m < min_ref[...]
    idx_ref[...] = jnp.where(upd, cand, idx_ref[...])
    min_ref[...] = jnp.where(upd, m, min_ref[...])


def _argmin_call(predp, predpt, gtpt):
    return pl.pallas_call(
        _argmin_body,
        grid=(NSTEPS,),
        in_specs=[
            pl.BlockSpec((NPRED, 3), lambda j: (0, 0)),
            pl.BlockSpec((3, NPRED), lambda j: (0, 0)),
            pl.BlockSpec((3, NGT), lambda j: (0, 0)),
            pl.BlockSpec((3, CHUNK), lambda j: (0, j)),
            pl.BlockSpec((1, CHUNK), lambda j: (0, 0)),
        ],
        out_specs=[
            pl.BlockSpec((NPRED, 1), lambda j: (0, 0)),
            pl.BlockSpec((3, 1), lambda j: (0, 0)),
            pl.BlockSpec((NPRED, 1), lambda j: (0, 0)),
        ],
        out_shape=[
            jax.ShapeDtypeStruct((NPRED, 1), jnp.int32),
            jax.ShapeDtypeStruct((3, 1), jnp.float32),
            jax.ShapeDtypeStruct((NPRED, 1), jnp.float32),
        ],
        scratch_shapes=[pltpu.VMEM((NPRED, CHUNK), jnp.float32)],
    )(predp, predpt, gtpt, gtpt,
      jnp.arange(CHUNK, dtype=jnp.float32).reshape(1, CHUNK))


@functools.cache
def _sc_gather_fn():
    mesh = plsc.VectorSubcoreMesh(
        core_axis_name="c", subcore_axis_name="s",
        num_cores=2, num_subcores=16)

    @functools.partial(
        pl.kernel,
        out_type=jax.ShapeDtypeStruct((NPRED, 128), jnp.float32),
        mesh=mesh,
        scratch_types=[
            pltpu.VMEM((BPW,), jnp.int32),
            pltpu.VMEM((BPW, 128), jnp.float32),
            pltpu.SemaphoreType.DMA,
        ],
    )
    def _sc_gather(table_hbm, idx_hbm, out_hbm, idx_v, rows_v, sem):
        wid = lax.axis_index("s") * 2 + lax.axis_index("c")
        base = wid * BPW
        pltpu.sync_copy(idx_hbm.at[pl.ds(base, BPW)], idx_v)
        pltpu.async_copy(table_hbm.at[idx_v], rows_v, sem).wait()
        pltpu.sync_copy(rows_v, out_hbm.at[pl.ds(base, BPW)])

    return _sc_gather


def _loss_body(pf_ref, mt_ref, trow_ref, out_ref):
    pf = pf_ref[...]
    mt = mt_ref[...]
    trow = trow_ref[...]

    d = (pf[:, 0:3] + trow) - mt[:, 0:3]
    ad = jnp.abs(d)
    h = jnp.where(ad < 1.0, 0.5 * ad * ad, ad - 0.5)
    huber = jnp.sum(h) / jnp.float32(NPRED * 3)
    reg = huber / 2000.0

    pn = pf[:, 3:6]
    gn = mt[:, 3:6]
    dp = jnp.maximum(jnp.sqrt(jnp.sum(pn * pn, axis=1, keepdims=True)), 1e-5)
    dg = jnp.maximum(jnp.sqrt(jnp.sum(gn * gn, axis=1, keepdims=True)), 1e-5)
    cos = jnp.sum(pn * gn, axis=1, keepdims=True) / (dp * dg)
    norm_loss = 1.0 - jnp.sum(cos) / jnp.float32(NPRED)

    out_ref[...] = jnp.broadcast_to(reg + norm_loss, (1, 1))


def _loss_call(pred_feat, matched, t):
    return pl.pallas_call(
        _loss_body,
        out_shape=jax.ShapeDtypeStruct((1, 1), jnp.float32),
    )(pred_feat, matched, t.reshape(1, 3))


def kernel(pred_feat, pred_decoder, input_data, gt_data):
    predp = pred_feat[:, :3]
    predpt = predp.T
    gtpt = gt_data[:, :3].T

    idx2d, t, _ = _argmin_call(predp, predpt, gtpt)
    idx = idx2d.reshape(NPRED)

    table = jnp.pad(gt_data, ((0, 0), (0, 122)))
    matched = _sc_gather_fn()(table, idx)

    out = _loss_call(pred_feat, matched, t)
    return out.reshape(1)
